# baseline (speedup 1.0000x reference)
import functools

import jax
import jax.numpy as jnp
import numpy as np
from jax import lax
from jax.experimental import pallas as pl
from jax.experimental.pallas import tpu as pltpu
from jax.experimental.pallas import tpu_sc as plsc

B = 16384
D = 64
NC = 2
NS = 16
NW = NC * NS
BPW = B // NW
CHUNK = 128
NCH = BPW // CHUNK

_INV_SQRT = float(1.0 / np.sqrt(1.0 + 1e-5))

@functools.cache
def _make_sc_gather():
    mesh = plsc.VectorSubcoreMesh(core_axis_name="c", subcore_axis_name="s")

    @functools.partial(
        pl.kernel,
        mesh=mesh,
        out_type=[
            jax.ShapeDtypeStruct((B, D), jnp.float32),
            jax.ShapeDtypeStruct((B, D), jnp.float32),
        ],
        scratch_types=[
            pltpu.VMEM((NCH, CHUNK), jnp.int32),
            pltpu.VMEM((NCH, CHUNK), jnp.int32),
            pltpu.VMEM((BPW, D), jnp.float32),
            pltpu.VMEM((BPW, D), jnp.float32),
            pltpu.SemaphoreType.DMA,
            pltpu.SemaphoreType.DMA,
        ],
        compiler_params=pltpu.CompilerParams(use_tc_tiling_on_sc=False),
    )
    def _sc_gather(uid_hbm, iid_hbm, ut_hbm, it_hbm, ue_hbm, ie_hbm,
                   uidx, iidx, urows, irows, sem_u, sem_i):
        wid = lax.axis_index("s") * NC + lax.axis_index("c")
        base = wid * BPW
        pltpu.sync_copy(uid_hbm.at[wid], uidx)
        pltpu.sync_copy(iid_hbm.at[wid], iidx)
        copies = []
        for j in range(NCH):
            copies.append(pltpu.async_copy(
                ut_hbm.at[uidx.at[j]], urows.at[pl.ds(j * CHUNK, CHUNK)], sem_u))
            copies.append(pltpu.async_copy(
                it_hbm.at[iidx.at[j]], irows.at[pl.ds(j * CHUNK, CHUNK)], sem_i))
        for c in copies:
            c.wait()
        pltpu.sync_copy(urows, ue_hbm.at[pl.ds(base, BPW)])
        pltpu.sync_copy(irows, ie_hbm.at[pl.ds(base, BPW)])

    return _sc_gather


BLK = 2048


def _mlp_body(ue_ref, ie_ref, w0_ref, b0_ref, g0_ref, bt0_ref,
              w1_ref, b1_ref, g1_ref, bt1_ref,
              w2_ref, b2_ref, g2_ref, bt2_ref,
              wo_ref, bo_ref, out_ref):
    x = (jnp.dot(ue_ref[...], w0_ref[0:D, :], preferred_element_type=jnp.float32)
         + jnp.dot(ie_ref[...], w0_ref[D:2 * D, :], preferred_element_type=jnp.float32)
         + b0_ref[...])
    x = jnp.maximum(x, 0.0) * (g0_ref[...] * _INV_SQRT) + bt0_ref[...]
    x = jnp.dot(x, w1_ref[...], preferred_element_type=jnp.float32) + b1_ref[...]
    x = jnp.maximum(x, 0.0) * (g1_ref[...] * _INV_SQRT) + bt1_ref[...]
    x = jnp.dot(x, w2_ref[...], preferred_element_type=jnp.float32) + b2_ref[...]
    x = jnp.maximum(x, 0.0) * (g2_ref[...] * _INV_SQRT) + bt2_ref[...]
    z = jnp.sum(x * wo_ref[...], axis=1) + bo_ref[0, 0]
    out_ref[...] = 1.0 / (1.0 + jnp.exp(-z))


def _full(shape):
    return pl.BlockSpec(shape, lambda i: (0,) * len(shape))


def kernel(user_ids, item_ids, user_table, item_table,
           W0, b0, gamma0, beta0,
           W1, b1, gamma1, beta1,
           W2, b2, gamma2, beta2,
           Wo, bo):
    uid3 = user_ids.astype(jnp.int32).reshape(NW, NCH, CHUNK)
    iid3 = item_ids.astype(jnp.int32).reshape(NW, NCH, CHUNK)
    ue, ie = _make_sc_gather()(uid3, iid3, user_table, item_table)

    grid = (B // BLK,)
    vec_specs = []
    ins = [ue, ie]
    in_specs = [
        pl.BlockSpec((BLK, D), lambda i: (i, 0)),
        pl.BlockSpec((BLK, D), lambda i: (i, 0)),
    ]
    for (W, b, g, bt) in ((W0, b0, gamma0, beta0), (W1, b1, gamma1, beta1),
                          (W2, b2, gamma2, beta2)):
        h = W.shape[1]
        ins += [W, b.reshape(1, h), g.reshape(1, h), bt.reshape(1, h)]
        in_specs += [_full(W.shape), _full((1, h)), _full((1, h)), _full((1, h))]
    ins += [Wo.reshape(1, Wo.shape[0]), bo.reshape(1, 1)]
    in_specs += [_full((1, Wo.shape[0])),
                 pl.BlockSpec(memory_space=pltpu.SMEM)]

    out = pl.pallas_call(
        _mlp_body,
        grid=grid,
        in_specs=in_specs,
        out_specs=pl.BlockSpec((BLK,), lambda i: (i,)),
        out_shape=jax.ShapeDtypeStruct((B,), jnp.float32),
    )(*ins)
    return out
